# Initial kernel scaffold; baseline (speedup 1.0000x reference)
import jax
import jax.numpy as jnp
from jax.experimental import pallas as pl


def kernel(proj, labels, core_prototypes, transition_prototypes, reliability_map):
    raise NotImplementedError("write your pallas kernel here")


# speedup vs baseline: 1.5666x; 1.5666x over previous
import functools

import jax
import jax.numpy as jnp
from jax.experimental import pallas as pl
from jax.experimental.pallas import tpu as pltpu

TEMP = 0.1
EPS = 1e-8


def _body(x_ref, lab_ref, rel_ref, pr_ref, out_ref, pn_ref, acc_ref, *, K, P, T):
    b = pl.program_id(0)
    s = pl.program_id(1)

    @pl.when(jnp.logical_and(b == 0, s == 0))
    def _init():
        p = pr_ref[...]
        pn_ref[...] = p / (jnp.sqrt(jnp.sum(p * p, axis=1, keepdims=True)) + EPS)
        acc_ref[0] = 0.0
        acc_ref[1] = 0.0

    x = x_ref[0]
    colsq = jnp.sum(x * x, axis=0, keepdims=True)
    inv = 1.0 / ((jnp.sqrt(colsq) + EPS) * TEMP)
    logits = jnp.dot(pn_ref[...], x, preferred_element_type=jnp.float32) * inv
    cl = jnp.max(logits.reshape(K, P, T), axis=1)
    m = jnp.max(cl, axis=0, keepdims=True)
    lse = m + jnp.log(jnp.sum(jnp.exp(cl - m), axis=0, keepdims=True))
    lab = lab_ref[0]
    kidx = jax.lax.broadcasted_iota(jnp.int32, (K, T), 0)
    label_logit = jnp.sum(jnp.where(kidx == lab, cl, 0.0), axis=0, keepdims=True)
    nll = lse - label_logit
    w = rel_ref[0]
    acc_ref[0] += jnp.sum(nll * w)
    acc_ref[1] += jnp.sum(w)

    @pl.when(jnp.logical_and(b == pl.num_programs(0) - 1,
                             s == pl.num_programs(1) - 1))
    def _fin():
        out_ref[0, 0] = acc_ref[0] / (acc_ref[1] + EPS)


def kernel(proj, labels, core_prototypes, transition_prototypes, reliability_map):
    B, C, H, W = proj.shape
    S = H * W
    K, Pc, _ = core_prototypes.shape
    P = Pc + transition_prototypes.shape[1]
    protos = jnp.concatenate([core_prototypes, transition_prototypes], axis=1)
    protos = protos.reshape(K * P, C)

    x = proj.reshape(B, C, S)
    lab = labels.reshape(B, 1, S)
    rel = reliability_map.reshape(B, 1, S)

    T = 2048
    grid = (B, S // T)

    out = pl.pallas_call(
        functools.partial(_body, K=K, P=P, T=T),
        grid=grid,
        in_specs=[
            pl.BlockSpec((1, C, T), lambda b, s: (b, 0, s)),
            pl.BlockSpec((1, 1, T), lambda b, s: (b, 0, s)),
            pl.BlockSpec((1, 1, T), lambda b, s: (b, 0, s)),
            pl.BlockSpec((K * P, C), lambda b, s: (0, 0)),
        ],
        out_specs=pl.BlockSpec((1, 1), lambda b, s: (0, 0),
                               memory_space=pltpu.SMEM),
        out_shape=jax.ShapeDtypeStruct((1, 1), jnp.float32),
        scratch_shapes=[
            pltpu.VMEM((K * P, C), jnp.float32),
            pltpu.SMEM((2,), jnp.float32),
        ],
    )(x, lab, rel, protos)
    return out.reshape(())


# speedup vs baseline: 1.6110x; 1.0284x over previous
import functools

import jax
import jax.numpy as jnp
from jax.experimental import pallas as pl
from jax.experimental.pallas import tpu as pltpu

TEMP = 0.1
EPS = 1e-8
KPAD = 24


def _body(x_ref, lab_ref, rel_ref, pr_ref, out_ref, pn_ref, acc_ref, *, K, P, T, C):
    s = pl.program_id(1)

    @pl.when(s == 0)
    def _init():
        p = pr_ref[...]
        pn_ref[...] = p / (jnp.sqrt(jnp.sum(p * p, axis=1, keepdims=True)) + EPS)
        acc_ref[0] = 0.0
        acc_ref[1] = 0.0

    x = x_ref[0]
    xsq = x * x
    colsq = jnp.dot(jnp.ones((1, C), jnp.float32), xsq,
                    preferred_element_type=jnp.float32)
    inv = 1.0 / ((jnp.sqrt(colsq) + EPS) * TEMP)
    mm = jnp.dot(pn_ref[...], x, preferred_element_type=jnp.float32)
    cl = jnp.max(mm.reshape(P, KPAD, T), axis=0) * inv
    kidx = jax.lax.broadcasted_iota(jnp.int32, (KPAD, T), 0)
    cl = jnp.where(kidx < K, cl, -1e4)
    lse = jnp.log(jnp.sum(jnp.exp(cl), axis=0, keepdims=True))
    lab = lab_ref[0]
    label_logit = jnp.sum(jnp.where(kidx == lab, cl, 0.0), axis=0, keepdims=True)
    nll = lse - label_logit
    w = rel_ref[0]
    acc_ref[0] += jnp.sum(nll * w)
    acc_ref[1] += jnp.sum(w)

    @pl.when(s == pl.num_programs(1) - 1)
    def _fin():
        out_ref[0, 0, 0] = acc_ref[0]
        out_ref[0, 0, 1] = acc_ref[1]


def kernel(proj, labels, core_prototypes, transition_prototypes, reliability_map):
    B, C, H, W = proj.shape
    S = H * W
    K, Pc, _ = core_prototypes.shape
    P = Pc + transition_prototypes.shape[1]
    protos = jnp.concatenate([core_prototypes, transition_prototypes], axis=1)
    protos = jnp.transpose(protos, (1, 0, 2))
    protos = jnp.pad(protos, ((0, 0), (0, KPAD - K), (0, 0)))
    protos = protos.reshape(P * KPAD, C)

    x = proj.reshape(B, C, S)
    lab = labels.reshape(B, 1, S)
    rel = reliability_map.reshape(B, 1, S)

    T = 2048
    grid = (B, S // T)

    part = pl.pallas_call(
        functools.partial(_body, K=K, P=P, T=T, C=C),
        grid=grid,
        in_specs=[
            pl.BlockSpec((1, C, T), lambda b, s: (b, 0, s)),
            pl.BlockSpec((1, 1, T), lambda b, s: (b, 0, s)),
            pl.BlockSpec((1, 1, T), lambda b, s: (b, 0, s)),
            pl.BlockSpec((P * KPAD, C), lambda b, s: (0, 0)),
        ],
        out_specs=pl.BlockSpec((1, 1, 2), lambda b, s: (b, 0, 0),
                               memory_space=pltpu.SMEM),
        out_shape=jax.ShapeDtypeStruct((B, 1, 2), jnp.float32),
        scratch_shapes=[
            pltpu.VMEM((P * KPAD, C), jnp.float32),
            pltpu.SMEM((2,), jnp.float32),
        ],
        compiler_params=pltpu.CompilerParams(
            dimension_semantics=("parallel", "arbitrary")),
    )(x, lab, rel, protos)
    return part[:, 0, 0].sum() / (part[:, 0, 1].sum() + EPS)


# speedup vs baseline: 1.7404x; 1.0803x over previous
import functools

import jax
import jax.numpy as jnp
from jax.experimental import pallas as pl
from jax.experimental.pallas import tpu as pltpu

TEMP = 0.1
EPS = 1e-8
KPAD = 24


def _body(x_ref, lab_ref, rel_ref, pr_ref, out_ref, pn_ref, acc_ref, *, K, P, T, C):
    s = pl.program_id(1)

    @pl.when(s == 0)
    def _init():
        p = pr_ref[...]
        pn_ref[...] = p / (jnp.sqrt(jnp.sum(p * p, axis=1, keepdims=True)) + EPS)
        acc_ref[0] = 0.0
        acc_ref[1] = 0.0

    x = x_ref[0]
    xsq = x * x
    colsq = jnp.dot(jnp.ones((1, C), jnp.float32), xsq,
                    preferred_element_type=jnp.float32)
    inv = 1.0 / ((jnp.sqrt(colsq) + EPS) * TEMP)
    mm = jnp.dot(pn_ref[...], x, preferred_element_type=jnp.float32)
    cl = jnp.max(mm.reshape(P, KPAD, T), axis=0) * inv
    kidx = jax.lax.broadcasted_iota(jnp.int32, (KPAD, T), 0)
    cl = jnp.where(kidx < K, cl, -1e4)
    lse = jnp.log(jnp.sum(jnp.exp(cl), axis=0, keepdims=True))
    lab = lab_ref[0]
    label_logit = jnp.sum(jnp.where(kidx == lab, cl, 0.0), axis=0, keepdims=True)
    nll = lse - label_logit
    w = rel_ref[0]
    acc_ref[0] += jnp.sum(nll * w)
    acc_ref[1] += jnp.sum(w)

    @pl.when(s == pl.num_programs(1) - 1)
    def _fin():
        out_ref[0, 0, 0] = acc_ref[0]
        out_ref[0, 0, 1] = acc_ref[1]


def kernel(proj, labels, core_prototypes, transition_prototypes, reliability_map):
    B, C, H, W = proj.shape
    S = H * W
    K, Pc, _ = core_prototypes.shape
    P = Pc + transition_prototypes.shape[1]
    protos = jnp.concatenate([core_prototypes, transition_prototypes], axis=1)
    protos = jnp.transpose(protos, (1, 0, 2))
    protos = jnp.pad(protos, ((0, 0), (0, KPAD - K), (0, 0)))
    protos = protos.reshape(P * KPAD, C)

    x = proj.reshape(B, C, S)
    lab = labels.reshape(B, 1, S)
    rel = reliability_map.reshape(B, 1, S)

    T = 4096
    grid = (B, S // T)

    part = pl.pallas_call(
        functools.partial(_body, K=K, P=P, T=T, C=C),
        grid=grid,
        in_specs=[
            pl.BlockSpec((1, C, T), lambda b, s: (b, 0, s)),
            pl.BlockSpec((1, 1, T), lambda b, s: (b, 0, s)),
            pl.BlockSpec((1, 1, T), lambda b, s: (b, 0, s)),
            pl.BlockSpec((P * KPAD, C), lambda b, s: (0, 0)),
        ],
        out_specs=pl.BlockSpec((1, 1, 2), lambda b, s: (b, 0, 0),
                               memory_space=pltpu.SMEM),
        out_shape=jax.ShapeDtypeStruct((B, 1, 2), jnp.float32),
        scratch_shapes=[
            pltpu.VMEM((P * KPAD, C), jnp.float32),
            pltpu.SMEM((2,), jnp.float32),
        ],
        compiler_params=pltpu.CompilerParams(
            dimension_semantics=("parallel", "arbitrary")),
    )(x, lab, rel, protos)
    return part[:, 0, 0].sum() / (part[:, 0, 1].sum() + EPS)


# speedup vs baseline: 1.8221x; 1.0470x over previous
import functools

import jax
import jax.numpy as jnp
from jax.experimental import pallas as pl
from jax.experimental.pallas import tpu as pltpu

TEMP = 0.1
EPS = 1e-8
KPAD = 24


def _body(x_ref, lab_ref, rel_ref, pr_ref, out_ref, pn_ref, acc_ref, *, K, P, T, C):
    s = pl.program_id(1)

    @pl.when(s == 0)
    def _init():
        p = pr_ref[...]
        pn_ref[...] = p / (jnp.sqrt(jnp.sum(p * p, axis=1, keepdims=True)) + EPS)
        acc_ref[0] = 0.0
        acc_ref[1] = 0.0

    x = x_ref[0]
    xsq = x * x
    colsq = jnp.dot(jnp.ones((1, C), jnp.float32), xsq,
                    preferred_element_type=jnp.float32)
    inv = 1.0 / ((jnp.sqrt(colsq) + EPS) * TEMP)
    mm = jnp.dot(pn_ref[...], x, preferred_element_type=jnp.float32)
    cl = jnp.max(mm.reshape(P, KPAD, T), axis=0) * inv
    kidx = jax.lax.broadcasted_iota(jnp.int32, (KPAD, T), 0)
    cl = jnp.where(kidx < K, cl, -1e4)
    lse = jnp.log(jnp.sum(jnp.exp(cl), axis=0, keepdims=True))
    lab = lab_ref[0]
    label_logit = jnp.sum(jnp.where(kidx == lab, cl, 0.0), axis=0, keepdims=True)
    nll = lse - label_logit
    w = rel_ref[0]
    acc_ref[0] += jnp.sum(nll * w)
    acc_ref[1] += jnp.sum(w)

    @pl.when(s == pl.num_programs(1) - 1)
    def _fin():
        out_ref[0, 0, 0] = acc_ref[0]
        out_ref[0, 0, 1] = acc_ref[1]


def kernel(proj, labels, core_prototypes, transition_prototypes, reliability_map):
    B, C, H, W = proj.shape
    S = H * W
    K, Pc, _ = core_prototypes.shape
    P = Pc + transition_prototypes.shape[1]
    protos = jnp.concatenate([core_prototypes, transition_prototypes], axis=1)
    protos = jnp.transpose(protos, (1, 0, 2))
    protos = jnp.pad(protos, ((0, 0), (0, KPAD - K), (0, 0)))
    protos = protos.reshape(P * KPAD, C)

    x = proj.reshape(B, C, S)
    lab = labels.reshape(B, 1, S)
    rel = reliability_map.reshape(B, 1, S)

    T = 8192
    grid = (B, S // T)

    part = pl.pallas_call(
        functools.partial(_body, K=K, P=P, T=T, C=C),
        grid=grid,
        in_specs=[
            pl.BlockSpec((1, C, T), lambda b, s: (b, 0, s)),
            pl.BlockSpec((1, 1, T), lambda b, s: (b, 0, s)),
            pl.BlockSpec((1, 1, T), lambda b, s: (b, 0, s)),
            pl.BlockSpec((P * KPAD, C), lambda b, s: (0, 0)),
        ],
        out_specs=pl.BlockSpec((1, 1, 2), lambda b, s: (b, 0, 0),
                               memory_space=pltpu.SMEM),
        out_shape=jax.ShapeDtypeStruct((B, 1, 2), jnp.float32),
        scratch_shapes=[
            pltpu.VMEM((P * KPAD, C), jnp.float32),
            pltpu.SMEM((2,), jnp.float32),
        ],
        compiler_params=pltpu.CompilerParams(
            dimension_semantics=("parallel", "arbitrary")),
    )(x, lab, rel, protos)
    return part[:, 0, 0].sum() / (part[:, 0, 1].sum() + EPS)


# speedup vs baseline: 3.5998x; 1.9757x over previous
import functools

import jax
import jax.numpy as jnp
from jax.experimental import pallas as pl
from jax.experimental.pallas import tpu as pltpu

TEMP = 0.1
EPS = 1e-8
KPAD = 24


def _body(x_ref, lab_ref, rel_ref, pr_ref, out_ref, pn_ref, acc_ref, *, K, P, T, C):
    s = pl.program_id(1)

    @pl.when(s == 0)
    def _init():
        p = pr_ref[...]
        pn_ref[...] = p / (jnp.sqrt(jnp.sum(p * p, axis=1, keepdims=True)) + EPS)
        acc_ref[0] = 0.0
        acc_ref[1] = 0.0

    x = x_ref[0].reshape(C, T)
    xsq = x * x
    colsq = jnp.dot(jnp.ones((1, C), jnp.float32), xsq,
                    preferred_element_type=jnp.float32)
    inv = 1.0 / ((jnp.sqrt(colsq) + EPS) * TEMP)
    mm = jnp.dot(pn_ref[...], x, preferred_element_type=jnp.float32)
    cl = jnp.max(mm.reshape(P, KPAD, T), axis=0) * inv
    kidx = jax.lax.broadcasted_iota(jnp.int32, (KPAD, T), 0)
    cl = jnp.where(kidx < K, cl, -1e4)
    lse = jnp.log(jnp.sum(jnp.exp(cl), axis=0, keepdims=True))
    lab = lab_ref[0]
    label_logit = jnp.sum(jnp.where(kidx == lab, cl, 0.0), axis=0, keepdims=True)
    nll = lse - label_logit
    w = rel_ref[0]
    acc_ref[0] += jnp.sum(nll * w)
    acc_ref[1] += jnp.sum(w)

    @pl.when(s == pl.num_programs(1) - 1)
    def _fin():
        out_ref[0, 0, 0] = acc_ref[0]
        out_ref[0, 0, 1] = acc_ref[1]


def kernel(proj, labels, core_prototypes, transition_prototypes, reliability_map):
    B, C, H, W = proj.shape
    S = H * W
    K, Pc, _ = core_prototypes.shape
    P = Pc + transition_prototypes.shape[1]
    protos = jnp.concatenate([core_prototypes, transition_prototypes], axis=1)
    protos = jnp.transpose(protos, (1, 0, 2))
    protos = jnp.pad(protos, ((0, 0), (0, KPAD - K), (0, 0)))
    protos = protos.reshape(P * KPAD, C)

    lab = labels.reshape(B, 1, S)
    rel = reliability_map.reshape(B, 1, S)

    Hb = 16
    T = Hb * W
    grid = (B, S // T)

    part = pl.pallas_call(
        functools.partial(_body, K=K, P=P, T=T, C=C),
        grid=grid,
        in_specs=[
            pl.BlockSpec((1, C, Hb, W), lambda b, s: (b, 0, s, 0)),
            pl.BlockSpec((1, 1, T), lambda b, s: (b, 0, s)),
            pl.BlockSpec((1, 1, T), lambda b, s: (b, 0, s)),
            pl.BlockSpec((P * KPAD, C), lambda b, s: (0, 0)),
        ],
        out_specs=pl.BlockSpec((1, 1, 2), lambda b, s: (b, 0, 0),
                               memory_space=pltpu.SMEM),
        out_shape=jax.ShapeDtypeStruct((B, 1, 2), jnp.float32),
        scratch_shapes=[
            pltpu.VMEM((P * KPAD, C), jnp.float32),
            pltpu.SMEM((2,), jnp.float32),
        ],
        compiler_params=pltpu.CompilerParams(
            dimension_semantics=("parallel", "arbitrary")),
    )(proj, lab, rel, protos)
    return part[:, 0, 0].sum() / (part[:, 0, 1].sum() + EPS)


# speedup vs baseline: 3.8326x; 1.0647x over previous
import functools

import jax
import jax.numpy as jnp
from jax.experimental import pallas as pl
from jax.experimental.pallas import tpu as pltpu

TEMP = 0.1
EPS = 1e-8
KPAD = 24


def _body(x_ref, lab_ref, rel_ref, pr_ref, out_ref, pn_ref, acc_ref, *, K, P, T, C):
    s = pl.program_id(1)

    @pl.when(s == 0)
    def _init():
        p = pr_ref[...]
        pn = p / (jnp.sqrt(jnp.sum(p * p, axis=1, keepdims=True)) + EPS)
        pn_ref[...] = pn.astype(jnp.bfloat16)
        acc_ref[0] = 0.0
        acc_ref[1] = 0.0

    x = x_ref[0].astype(jnp.bfloat16).reshape(C, T)
    xsq = x * x
    colsq = jnp.dot(jnp.ones((1, C), jnp.bfloat16), xsq,
                    preferred_element_type=jnp.float32)
    inv = 1.0 / ((jnp.sqrt(colsq) + EPS) * TEMP)
    mm = jnp.dot(pn_ref[...], x, preferred_element_type=jnp.float32)
    cl = jnp.max(mm.reshape(P, KPAD, T), axis=0) * inv
    kidx = jax.lax.broadcasted_iota(jnp.int32, (KPAD, T), 0)
    cl = jnp.where(kidx < K, cl, -1e4)
    lse = jnp.log(jnp.sum(jnp.exp(cl), axis=0, keepdims=True))
    lab = lab_ref[0]
    label_logit = jnp.sum(jnp.where(kidx == lab, cl, 0.0), axis=0, keepdims=True)
    nll = lse - label_logit
    w = rel_ref[0]
    acc_ref[0] += jnp.sum(nll * w)
    acc_ref[1] += jnp.sum(w)

    @pl.when(s == pl.num_programs(1) - 1)
    def _fin():
        out_ref[0, 0, 0] = acc_ref[0]
        out_ref[0, 0, 1] = acc_ref[1]


def kernel(proj, labels, core_prototypes, transition_prototypes, reliability_map):
    B, C, H, W = proj.shape
    S = H * W
    K, Pc, _ = core_prototypes.shape
    P = Pc + transition_prototypes.shape[1]
    protos = jnp.concatenate([core_prototypes, transition_prototypes], axis=1)
    protos = jnp.transpose(protos, (1, 0, 2))
    protos = jnp.pad(protos, ((0, 0), (0, KPAD - K), (0, 0)))
    protos = protos.reshape(P * KPAD, C)

    lab = labels.reshape(B, 1, S)
    rel = reliability_map.reshape(B, 1, S)

    Hb = 16
    T = Hb * W
    grid = (B, S // T)

    part = pl.pallas_call(
        functools.partial(_body, K=K, P=P, T=T, C=C),
        grid=grid,
        in_specs=[
            pl.BlockSpec((1, C, Hb, W), lambda b, s: (b, 0, s, 0)),
            pl.BlockSpec((1, 1, T), lambda b, s: (b, 0, s)),
            pl.BlockSpec((1, 1, T), lambda b, s: (b, 0, s)),
            pl.BlockSpec((P * KPAD, C), lambda b, s: (0, 0)),
        ],
        out_specs=pl.BlockSpec((1, 1, 2), lambda b, s: (b, 0, 0),
                               memory_space=pltpu.SMEM),
        out_shape=jax.ShapeDtypeStruct((B, 1, 2), jnp.float32),
        scratch_shapes=[
            pltpu.VMEM((P * KPAD, C), jnp.bfloat16),
            pltpu.SMEM((2,), jnp.float32),
        ],
        compiler_params=pltpu.CompilerParams(
            dimension_semantics=("parallel", "arbitrary")),
    )(proj, lab, rel, protos)
    return part[:, 0, 0].sum() / (part[:, 0, 1].sum() + EPS)


# speedup vs baseline: 4.7820x; 1.2477x over previous
import functools

import jax
import jax.numpy as jnp
from jax.experimental import pallas as pl
from jax.experimental.pallas import tpu as pltpu

TEMP = 0.1
EPS = 1e-8
KPAD = 24


def _body(x_ref, lab_ref, rel_ref, pr_ref, out_ref, pn_ref, acc_ref, *, K, P, T, C):
    s = pl.program_id(1)

    @pl.when(s == 0)
    def _init():
        p = pr_ref[...]
        pn = p / (jnp.sqrt(jnp.sum(p * p, axis=1, keepdims=True)) + EPS)
        pn_ref[...] = pn.astype(jnp.bfloat16)
        acc_ref[0] = 0.0
        acc_ref[1] = 0.0

    x = x_ref[0].astype(jnp.bfloat16).reshape(C, T)
    xsq = x * x
    colsq = jnp.dot(jnp.ones((1, C), jnp.bfloat16), xsq,
                    preferred_element_type=jnp.float32)
    inv = 1.0 / ((jnp.sqrt(colsq) + EPS) * TEMP)
    mm = jnp.dot(pn_ref[...], x, preferred_element_type=jnp.float32)
    cl = jnp.max(mm.reshape(P, KPAD, T), axis=0) * inv
    kidx = jax.lax.broadcasted_iota(jnp.int32, (KPAD, T), 0)
    cl = jnp.where(kidx < K, cl, -1e4)
    lse = jnp.log(jnp.sum(jnp.exp(cl), axis=0, keepdims=True))
    lab = lab_ref[0]
    label_logit = jnp.sum(jnp.where(kidx == lab, cl, 0.0), axis=0, keepdims=True)
    nll = lse - label_logit
    w = rel_ref[0]
    acc_ref[0] += jnp.sum(nll * w)
    acc_ref[1] += jnp.sum(w)

    @pl.when(s == pl.num_programs(1) - 1)
    def _fin():
        out_ref[0, 0, 0] = acc_ref[0]
        out_ref[0, 0, 1] = acc_ref[1]


def kernel(proj, labels, core_prototypes, transition_prototypes, reliability_map):
    B, C, H, W = proj.shape
    S = H * W
    K, Pc, _ = core_prototypes.shape
    P = Pc + transition_prototypes.shape[1]
    protos = jnp.concatenate([core_prototypes, transition_prototypes], axis=1)
    protos = jnp.transpose(protos, (1, 0, 2))
    protos = jnp.pad(protos, ((0, 0), (0, KPAD - K), (0, 0)))
    protos = protos.reshape(P * KPAD, C)

    lab = labels.reshape(B, 1, S)
    rel = reliability_map.reshape(B, 1, S)

    Hb = 32
    T = Hb * W
    grid = (B, S // T)

    part = pl.pallas_call(
        functools.partial(_body, K=K, P=P, T=T, C=C),
        grid=grid,
        in_specs=[
            pl.BlockSpec((1, C, Hb, W), lambda b, s: (b, 0, s, 0)),
            pl.BlockSpec((1, 1, T), lambda b, s: (b, 0, s)),
            pl.BlockSpec((1, 1, T), lambda b, s: (b, 0, s)),
            pl.BlockSpec((P * KPAD, C), lambda b, s: (0, 0)),
        ],
        out_specs=pl.BlockSpec((1, 1, 2), lambda b, s: (b, 0, 0),
                               memory_space=pltpu.SMEM),
        out_shape=jax.ShapeDtypeStruct((B, 1, 2), jnp.float32),
        scratch_shapes=[
            pltpu.VMEM((P * KPAD, C), jnp.bfloat16),
            pltpu.SMEM((2,), jnp.float32),
        ],
        compiler_params=pltpu.CompilerParams(
            dimension_semantics=("parallel", "arbitrary")),
    )(proj, lab, rel, protos)
    return part[:, 0, 0].sum() / (part[:, 0, 1].sum() + EPS)


# speedup vs baseline: 5.2725x; 1.1026x over previous
import functools

import jax
import jax.numpy as jnp
from jax.experimental import pallas as pl
from jax.experimental.pallas import tpu as pltpu

TEMP = 0.1
EPS = 1e-8
KPAD = 24


def _body(x_ref, lab_ref, rel_ref, pr_ref, out_ref, pn_ref, acc_ref, *, K, P, T, C):
    s = pl.program_id(1)

    @pl.when(s == 0)
    def _init():
        p = pr_ref[...]
        pn = p / (jnp.sqrt(jnp.sum(p * p, axis=1, keepdims=True)) + EPS)
        pn_ref[...] = pn.astype(jnp.bfloat16)
        acc_ref[0] = 0.0
        acc_ref[1] = 0.0

    x = x_ref[0].astype(jnp.bfloat16).reshape(C, T)
    xsq = x * x
    colsq = jnp.dot(jnp.ones((1, C), jnp.bfloat16), xsq,
                    preferred_element_type=jnp.float32)
    inv = 1.0 / ((jnp.sqrt(colsq) + EPS) * TEMP)
    mm = jnp.dot(pn_ref[...], x, preferred_element_type=jnp.float32)
    cl = jnp.max(mm.reshape(P, KPAD, T), axis=0) * inv
    kidx = jax.lax.broadcasted_iota(jnp.int32, (KPAD, T), 0)
    cl = jnp.where(kidx < K, cl, -1e4)
    lse = jnp.log(jnp.sum(jnp.exp(cl), axis=0, keepdims=True))
    lab = lab_ref[0]
    label_logit = jnp.sum(jnp.where(kidx == lab, cl, 0.0), axis=0, keepdims=True)
    nll = lse - label_logit
    w = rel_ref[0]
    acc_ref[0] += jnp.sum(nll * w)
    acc_ref[1] += jnp.sum(w)

    @pl.when(s == pl.num_programs(1) - 1)
    def _fin():
        out_ref[0, 0, 0] = acc_ref[0]
        out_ref[0, 0, 1] = acc_ref[1]


def kernel(proj, labels, core_prototypes, transition_prototypes, reliability_map):
    B, C, H, W = proj.shape
    S = H * W
    K, Pc, _ = core_prototypes.shape
    P = Pc + transition_prototypes.shape[1]
    protos = jnp.concatenate([core_prototypes, transition_prototypes], axis=1)
    protos = jnp.transpose(protos, (1, 0, 2))
    protos = jnp.pad(protos, ((0, 0), (0, KPAD - K), (0, 0)))
    protos = protos.reshape(P * KPAD, C)

    lab = labels.reshape(B, 1, S)
    rel = reliability_map.reshape(B, 1, S)

    Hb = 64
    T = Hb * W
    grid = (B, S // T)

    part = pl.pallas_call(
        functools.partial(_body, K=K, P=P, T=T, C=C),
        grid=grid,
        in_specs=[
            pl.BlockSpec((1, C, Hb, W), lambda b, s: (b, 0, s, 0)),
            pl.BlockSpec((1, 1, T), lambda b, s: (b, 0, s)),
            pl.BlockSpec((1, 1, T), lambda b, s: (b, 0, s)),
            pl.BlockSpec((P * KPAD, C), lambda b, s: (0, 0)),
        ],
        out_specs=pl.BlockSpec((1, 1, 2), lambda b, s: (b, 0, 0),
                               memory_space=pltpu.SMEM),
        out_shape=jax.ShapeDtypeStruct((B, 1, 2), jnp.float32),
        scratch_shapes=[
            pltpu.VMEM((P * KPAD, C), jnp.bfloat16),
            pltpu.SMEM((2,), jnp.float32),
        ],
        compiler_params=pltpu.CompilerParams(
            dimension_semantics=("parallel", "arbitrary")),
    )(proj, lab, rel, protos)
    return part[:, 0, 0].sum() / (part[:, 0, 1].sum() + EPS)


# speedup vs baseline: 5.3247x; 1.0099x over previous
import functools

import jax
import jax.numpy as jnp
from jax.experimental import pallas as pl
from jax.experimental.pallas import tpu as pltpu

TEMP = 0.1
EPS = 1e-8
KPAD = 24


def _body(x_ref, lab_ref, rel_ref, pr_ref, out_ref, pn_ref, acc_ref, *, K, P, T, C):
    s = pl.program_id(1)

    @pl.when(s == 0)
    def _init():
        p = pr_ref[...]
        pn = p / (jnp.sqrt(jnp.sum(p * p, axis=1, keepdims=True)) + EPS)
        pn_ref[...] = pn.astype(jnp.bfloat16)
        acc_ref[0] = 0.0
        acc_ref[1] = 0.0

    x = x_ref[0].astype(jnp.bfloat16).reshape(C, T)
    xsq = x * x
    colsq = jnp.dot(jnp.ones((1, C), jnp.bfloat16), xsq,
                    preferred_element_type=jnp.float32)
    inv = 1.0 / ((jnp.sqrt(colsq) + EPS) * TEMP)
    mm = jnp.dot(pn_ref[...], x, preferred_element_type=jnp.float32)
    cl = jnp.max(mm.reshape(P, KPAD, T), axis=0) * inv
    kidx = jax.lax.broadcasted_iota(jnp.int32, (KPAD, T), 0)
    cl = jnp.where(kidx < K, cl, -1e4)
    lse = jnp.log(jnp.sum(jnp.exp(cl), axis=0, keepdims=True))
    lab = lab_ref[0]
    label_logit = jnp.sum(jnp.where(kidx == lab, cl, 0.0), axis=0, keepdims=True)
    nll = lse - label_logit
    w = rel_ref[0]
    acc_ref[0] += jnp.sum(nll * w)
    acc_ref[1] += jnp.sum(w)

    @pl.when(s == pl.num_programs(1) - 1)
    def _fin():
        out_ref[0, 0, 0] = acc_ref[0]
        out_ref[0, 0, 1] = acc_ref[1]


def kernel(proj, labels, core_prototypes, transition_prototypes, reliability_map):
    B, C, H, W = proj.shape
    S = H * W
    K, Pc, _ = core_prototypes.shape
    P = Pc + transition_prototypes.shape[1]
    protos = jnp.concatenate([core_prototypes, transition_prototypes], axis=1)
    protos = jnp.transpose(protos, (1, 0, 2))
    protos = jnp.pad(protos, ((0, 0), (0, KPAD - K), (0, 0)))
    protos = protos.reshape(P * KPAD, C)

    lab = labels.reshape(B, 1, S)
    rel = reliability_map.reshape(B, 1, S)

    Hb = 128
    T = Hb * W
    grid = (B, S // T)

    part = pl.pallas_call(
        functools.partial(_body, K=K, P=P, T=T, C=C),
        grid=grid,
        in_specs=[
            pl.BlockSpec((1, C, Hb, W), lambda b, s: (b, 0, s, 0)),
            pl.BlockSpec((1, 1, T), lambda b, s: (b, 0, s)),
            pl.BlockSpec((1, 1, T), lambda b, s: (b, 0, s)),
            pl.BlockSpec((P * KPAD, C), lambda b, s: (0, 0)),
        ],
        out_specs=pl.BlockSpec((1, 1, 2), lambda b, s: (b, 0, 0),
                               memory_space=pltpu.SMEM),
        out_shape=jax.ShapeDtypeStruct((B, 1, 2), jnp.float32),
        scratch_shapes=[
            pltpu.VMEM((P * KPAD, C), jnp.bfloat16),
            pltpu.SMEM((2,), jnp.float32),
        ],
        compiler_params=pltpu.CompilerParams(
            dimension_semantics=("parallel", "arbitrary")),
    )(proj, lab, rel, protos)
    return part[:, 0, 0].sum() / (part[:, 0, 1].sum() + EPS)


# speedup vs baseline: 5.3309x; 1.0012x over previous
import functools

import jax
import jax.numpy as jnp
from jax.experimental import pallas as pl
from jax.experimental.pallas import tpu as pltpu

TEMP = 0.1
EPS = 1e-8
KPAD = 24


def _body(x0_ref, x1_ref, lab_ref, rel_ref, pr_ref, out_ref, pn_ref, acc_ref,
          *, K, P, T, C):
    s = pl.program_id(1)

    @pl.when(s == 0)
    def _init():
        p = pr_ref[...]
        pn = p / (jnp.sqrt(jnp.sum(p * p, axis=1, keepdims=True)) + EPS)
        pn_ref[...] = pn.astype(jnp.bfloat16)
        acc_ref[0] = 0.0
        acc_ref[1] = 0.0

    x = jnp.concatenate(
        [r[0].astype(jnp.bfloat16).reshape(C // 2, T) for r in (x0_ref, x1_ref)],
        axis=0)
    xsq = x * x
    colsq = jnp.dot(jnp.ones((1, C), jnp.bfloat16), xsq,
                    preferred_element_type=jnp.float32)
    inv = 1.0 / ((jnp.sqrt(colsq) + EPS) * TEMP)
    mm = jnp.dot(pn_ref[...], x, preferred_element_type=jnp.float32)
    cl = jnp.max(mm.reshape(P, KPAD, T), axis=0) * inv
    kidx = jax.lax.broadcasted_iota(jnp.int32, (KPAD, T), 0)
    cl = jnp.where(kidx < K, cl, -1e4)
    lse = jnp.log(jnp.sum(jnp.exp(cl), axis=0, keepdims=True))
    lab = lab_ref[0]
    label_logit = jnp.sum(jnp.where(kidx == lab, cl, 0.0), axis=0, keepdims=True)
    nll = lse - label_logit
    w = rel_ref[0]
    acc_ref[0] += jnp.sum(nll * w)
    acc_ref[1] += jnp.sum(w)

    @pl.when(s == pl.num_programs(1) - 1)
    def _fin():
        out_ref[0, 0, 0] = acc_ref[0]
        out_ref[0, 0, 1] = acc_ref[1]


def kernel(proj, labels, core_prototypes, transition_prototypes, reliability_map):
    B, C, H, W = proj.shape
    S = H * W
    K, Pc, _ = core_prototypes.shape
    P = Pc + transition_prototypes.shape[1]
    protos = jnp.concatenate([core_prototypes, transition_prototypes], axis=1)
    protos = jnp.transpose(protos, (1, 0, 2))
    protos = jnp.pad(protos, ((0, 0), (0, KPAD - K), (0, 0)))
    protos = protos.reshape(P * KPAD, C)

    lab = labels.reshape(B, 1, S)
    rel = reliability_map.reshape(B, 1, S)

    Hb = 128
    T = Hb * W
    grid = (B, S // T)

    part = pl.pallas_call(
        functools.partial(_body, K=K, P=P, T=T, C=C),
        grid=grid,
        in_specs=[
            pl.BlockSpec((1, C // 2, Hb, W), lambda b, s: (b, 0, s, 0)),
            pl.BlockSpec((1, C // 2, Hb, W), lambda b, s: (b, 1, s, 0)),
            pl.BlockSpec((1, 1, T), lambda b, s: (b, 0, s)),
            pl.BlockSpec((1, 1, T), lambda b, s: (b, 0, s)),
            pl.BlockSpec((P * KPAD, C), lambda b, s: (0, 0)),
        ],
        out_specs=pl.BlockSpec((1, 1, 2), lambda b, s: (b, 0, 0),
                               memory_space=pltpu.SMEM),
        out_shape=jax.ShapeDtypeStruct((B, 1, 2), jnp.float32),
        scratch_shapes=[
            pltpu.VMEM((P * KPAD, C), jnp.bfloat16),
            pltpu.SMEM((2,), jnp.float32),
        ],
        compiler_params=pltpu.CompilerParams(
            dimension_semantics=("parallel", "arbitrary")),
    )(proj, proj, lab, rel, protos)
    return part[:, 0, 0].sum() / (part[:, 0, 1].sum() + EPS)


# speedup vs baseline: 6.2156x; 1.1660x over previous
import functools

import jax
import jax.numpy as jnp
from jax.experimental import pallas as pl
from jax.experimental.pallas import tpu as pltpu

TEMP = 0.1
EPS = 1e-8
KPAD = 24


def _body(x_ref, lab_ref, rel_ref, core_ref, tr_ref, out_ref, pn_ref, acc_ref,
          *, K, P, T, C):
    b = pl.program_id(0)
    s = pl.program_id(1)

    @pl.when(jnp.logical_and(b == 0, s == 0))
    def _init():
        cc = jnp.concatenate([core_ref[...], tr_ref[...]], axis=1)
        cc = cc / (jnp.sqrt(jnp.sum(cc * cc, axis=2, keepdims=True)) + EPS)
        pn_ref[:, :K, :] = jnp.transpose(cc, (1, 0, 2)).astype(jnp.bfloat16)
        acc_ref[0] = 0.0
        acc_ref[1] = 0.0

    x = x_ref[0].astype(jnp.bfloat16).reshape(C, T)
    xsq = x * x
    colsq = jnp.dot(jnp.ones((1, C), jnp.bfloat16), xsq,
                    preferred_element_type=jnp.float32)
    inv = 1.0 / ((jnp.sqrt(colsq) + EPS) * TEMP)
    mm = jnp.dot(pn_ref[...].reshape(P * KPAD, C), x,
                 preferred_element_type=jnp.float32)
    cl = jnp.max(mm.reshape(P, KPAD, T), axis=0) * inv
    kidx = jax.lax.broadcasted_iota(jnp.int32, (KPAD, T), 0)
    cl = jnp.where(kidx < K, cl, -1e4)
    lse = jnp.log(jnp.sum(jnp.exp(cl), axis=0, keepdims=True))
    lab = lab_ref[0]
    label_logit = jnp.sum(jnp.where(kidx == lab, cl, 0.0), axis=0, keepdims=True)
    nll = lse - label_logit
    w = rel_ref[0]
    acc_ref[0] += jnp.sum(nll * w)
    acc_ref[1] += jnp.sum(w)

    @pl.when(jnp.logical_and(b == pl.num_programs(0) - 1,
                             s == pl.num_programs(1) - 1))
    def _fin():
        out_ref[0, 0] = acc_ref[0] / (acc_ref[1] + EPS)


def kernel(proj, labels, core_prototypes, transition_prototypes, reliability_map):
    B, C, H, W = proj.shape
    S = H * W
    K, Pc, _ = core_prototypes.shape
    P = Pc + transition_prototypes.shape[1]

    lab = labels.reshape(B, 1, S)
    rel = reliability_map.reshape(B, 1, S)

    Hb = 128
    T = Hb * W
    grid = (B, S // T)

    out = pl.pallas_call(
        functools.partial(_body, K=K, P=P, T=T, C=C),
        grid=grid,
        in_specs=[
            pl.BlockSpec((1, C, Hb, W), lambda b, s: (b, 0, s, 0)),
            pl.BlockSpec((1, 1, T), lambda b, s: (b, 0, s)),
            pl.BlockSpec((1, 1, T), lambda b, s: (b, 0, s)),
            pl.BlockSpec((K, Pc, C), lambda b, s: (0, 0, 0)),
            pl.BlockSpec((K, P - Pc, C), lambda b, s: (0, 0, 0)),
        ],
        out_specs=pl.BlockSpec((1, 1), lambda b, s: (0, 0),
                               memory_space=pltpu.SMEM),
        out_shape=jax.ShapeDtypeStruct((1, 1), jnp.float32),
        scratch_shapes=[
            pltpu.VMEM((P, KPAD, C), jnp.bfloat16),
            pltpu.SMEM((2,), jnp.float32),
        ],
    )(proj, lab, rel, core_prototypes, transition_prototypes)
    return out.reshape(())
